# baseline (speedup 1.0000x reference)
import functools

import jax
import jax.numpy as jnp
from jax import lax
from jax.experimental import pallas as pl
from jax.experimental.pallas import tpu as pltpu
from jax.experimental.pallas import tpu_sc as plsc

_N = 4096
_N_DEPTH = 3
_N_ORI = 100
_N_GEN = 10
_N_BINS = 50
_NPLANE = _N_DEPTH * _N_GEN
_NW = 32
_DOCS_PER_W = _N // _NW
_NQ = _DOCS_PER_W // 16
_TAB = 160


def _splat(ref, i):
    return plsc.load_gather(ref, [jnp.full((16,), i, jnp.int32)])


def _sc_body(planes_h, wm_h, wf_h, out0_h, out1_h,
             wtab, small, buf0, buf1, row0, row1, sem0, sem1):
    c = lax.axis_index("c")
    s = lax.axis_index("s")
    wid = s * 2 + c
    col = wid * _DOCS_PER_W

    pltpu.sync_copy(wm_h, wtab)
    pltpu.sync_copy(wf_h, small)

    bufs = (buf0, buf1)
    sems = (sem0, sem1)
    handles = [None, None]
    handles[0] = pltpu.async_copy(
        planes_h.at[0, 0, slice(None), pl.ds(col, _DOCS_PER_W)],
        bufs[0], sems[0])

    accs = tuple(jnp.zeros((16,), jnp.float32) for _ in range(_NQ))

    for p in range(_NPLANE):
        b = p & 1
        handles[b].wait()
        if p + 1 < _NPLANE:
            d1, g1 = divmod(p + 1, _N_GEN)
            handles[1 - b] = pltpu.async_copy(
                planes_h.at[d1, g1, slice(None), pl.ds(col, _DOCS_PER_W)],
                bufs[1 - b], sems[1 - b])

        buf = bufs[b]
        off = (p // _N_GEN) * _N_BINS
        pz = tuple(jnp.zeros((16,), jnp.float32) for _ in range(_NQ))

        @plsc.parallel_loop(0, _N_ORI, unroll=2, carry=pz)
        def body(r, acc_t, _buf=buf, _off=off):
            new = []
            for q in range(_NQ):
                x = _buf[r, pl.ds(16 * q, 16)]
                t = jnp.minimum(x * jnp.float32(_N_BINS),
                                jnp.float32(_N_BINS - 1))
                ti = t.astype(jnp.int32)
                if _off:
                    ti = ti + _off
                w = plsc.load_gather(wtab, [ti])
                new.append(acc_t[q] + w)
            return tuple(new)

        accs = tuple(a + pa for a, pa in zip(accs, body))

    sv = small[pl.ds(0, 16)]
    ws0 = sv[0] + sv[2] + sv[4]
    ws1 = sv[1] + sv[3] + sv[5]
    a0 = jnp.full((16,), ws0 * jnp.float32(1.0 / _N_GEN), jnp.float32)
    a1 = jnp.full((16,), ws1 * jnp.float32(1.0 / _N_GEN), jnp.float32)
    c0 = jnp.full((16,), sv[8] * ws0 + sv[6], jnp.float32)
    c1 = jnp.full((16,), sv[8] * ws1 + sv[7], jnp.float32)
    for q in range(_NQ):
        row0[pl.ds(q * 16, 16)] = accs[q] * a0 + c0
        row1[pl.ds(q * 16, 16)] = accs[q] * a1 + c1

    pltpu.sync_copy(row0, out0_h.at[pl.ds(col, _DOCS_PER_W)])
    pltpu.sync_copy(row1, out1_h.at[pl.ds(col, _DOCS_PER_W)])


_mesh = plsc.VectorSubcoreMesh(core_axis_name="c", subcore_axis_name="s")

_sc_call = functools.partial(
    pl.kernel,
    mesh=_mesh,
    compiler_params=pltpu.CompilerParams(needs_layout_passes=False),
    out_type=[
        jax.ShapeDtypeStruct((_N,), jnp.float32),
        jax.ShapeDtypeStruct((_N,), jnp.float32),
    ],
    scratch_types=[
        pltpu.VMEM((_TAB,), jnp.float32),
        pltpu.VMEM((16,), jnp.float32),
        pltpu.VMEM((_N_ORI, _DOCS_PER_W), jnp.float32),
        pltpu.VMEM((_N_ORI, _DOCS_PER_W), jnp.float32),
        pltpu.VMEM((_DOCS_PER_W,), jnp.float32),
        pltpu.VMEM((_DOCS_PER_W,), jnp.float32),
        pltpu.SemaphoreType.DMA,
        pltpu.SemaphoreType.DMA,
    ],
)(_sc_body)


def kernel(images, W_mlp, b_mlp, W_final, b_final):
    planes = jnp.transpose(images, (1, 3, 2, 0))
    wtab = jnp.concatenate(
        [W_mlp[:, 0], jnp.zeros((_TAB - _N_DEPTH * _N_BINS,), jnp.float32)])
    small = jnp.concatenate(
        [W_final.reshape(6), b_final, b_mlp,
         jnp.zeros((7,), jnp.float32)])
    out0, out1 = _sc_call(planes, wtab, small)
    return jnp.stack([out0, out1], axis=-1)


# speedup vs baseline: 1.0116x; 1.0116x over previous
import functools

import jax
import jax.numpy as jnp
from jax import lax
from jax.experimental import pallas as pl
from jax.experimental.pallas import tpu as pltpu
from jax.experimental.pallas import tpu_sc as plsc

_N = 4096
_N_DEPTH = 3
_N_ORI = 100
_N_GEN = 10
_N_BINS = 50
_NPLANE = _N_DEPTH * _N_GEN
_NW = 32
_DOCS_PER_W = _N // _NW
_NQ = _DOCS_PER_W // 16
_TAB = 160


def _splat(ref, i):
    return plsc.load_gather(ref, [jnp.full((16,), i, jnp.int32)])


def _sc_body(planes_h, wm_h, wf_h, out0_h, out1_h,
             wtab, small, buf0, buf1, row0, row1, sem0, sem1):
    c = lax.axis_index("c")
    s = lax.axis_index("s")
    wid = s * 2 + c
    col = wid * _DOCS_PER_W

    pltpu.sync_copy(wm_h, wtab)
    pltpu.sync_copy(wf_h, small)

    bufs = (buf0, buf1)
    sems = (sem0, sem1)
    handles = [None, None]
    handles[0] = pltpu.async_copy(
        planes_h.at[0, 0, slice(None), pl.ds(col, _DOCS_PER_W)],
        bufs[0], sems[0])

    accs = tuple(jnp.zeros((16,), jnp.float32) for _ in range(_NQ))

    for p in range(_NPLANE):
        b = p & 1
        handles[b].wait()
        if p + 1 < _NPLANE:
            d1, g1 = divmod(p + 1, _N_GEN)
            handles[1 - b] = pltpu.async_copy(
                planes_h.at[d1, g1, slice(None), pl.ds(col, _DOCS_PER_W)],
                bufs[1 - b], sems[1 - b])

        buf = bufs[b]
        off = (p // _N_GEN) * _N_BINS

        @plsc.parallel_loop(0, _N_ORI, unroll=2, carry=accs)
        def body(r, acc_t, _buf=buf, _off=off):
            new = []
            for q in range(_NQ):
                x = _buf[r, pl.ds(16 * q, 16)]
                t = jnp.minimum(x * jnp.float32(_N_BINS),
                                jnp.float32(_N_BINS - 1))
                ti = t.astype(jnp.int32)
                if _off:
                    ti = ti + _off
                w = plsc.load_gather(wtab, [ti])
                new.append(acc_t[q] + w)
            return tuple(new)

        accs = body

    sv = small[pl.ds(0, 16)]
    ws0 = sv[0] + sv[2] + sv[4]
    ws1 = sv[1] + sv[3] + sv[5]
    a0 = jnp.full((16,), ws0 * jnp.float32(1.0 / _N_GEN), jnp.float32)
    a1 = jnp.full((16,), ws1 * jnp.float32(1.0 / _N_GEN), jnp.float32)
    c0 = jnp.full((16,), sv[8] * ws0 + sv[6], jnp.float32)
    c1 = jnp.full((16,), sv[8] * ws1 + sv[7], jnp.float32)
    for q in range(_NQ):
        row0[pl.ds(q * 16, 16)] = accs[q] * a0 + c0
        row1[pl.ds(q * 16, 16)] = accs[q] * a1 + c1

    pltpu.sync_copy(row0, out0_h.at[pl.ds(col, _DOCS_PER_W)])
    pltpu.sync_copy(row1, out1_h.at[pl.ds(col, _DOCS_PER_W)])


_mesh = plsc.VectorSubcoreMesh(core_axis_name="c", subcore_axis_name="s")

_sc_call = functools.partial(
    pl.kernel,
    mesh=_mesh,
    compiler_params=pltpu.CompilerParams(needs_layout_passes=False),
    out_type=[
        jax.ShapeDtypeStruct((_N,), jnp.float32),
        jax.ShapeDtypeStruct((_N,), jnp.float32),
    ],
    scratch_types=[
        pltpu.VMEM((_TAB,), jnp.float32),
        pltpu.VMEM((16,), jnp.float32),
        pltpu.VMEM((_N_ORI, _DOCS_PER_W), jnp.float32),
        pltpu.VMEM((_N_ORI, _DOCS_PER_W), jnp.float32),
        pltpu.VMEM((_DOCS_PER_W,), jnp.float32),
        pltpu.VMEM((_DOCS_PER_W,), jnp.float32),
        pltpu.SemaphoreType.DMA,
        pltpu.SemaphoreType.DMA,
    ],
)(_sc_body)


def kernel(images, W_mlp, b_mlp, W_final, b_final):
    planes = jnp.transpose(images, (1, 3, 2, 0))
    wtab = jnp.concatenate(
        [W_mlp[:, 0], jnp.zeros((_TAB - _N_DEPTH * _N_BINS,), jnp.float32)])
    small = jnp.concatenate(
        [W_final.reshape(6), b_final, b_mlp,
         jnp.zeros((7,), jnp.float32)])
    out0, out1 = _sc_call(planes, wtab, small)
    return jnp.stack([out0, out1], axis=-1)
